# baseline (speedup 1.0000x reference)
import jax
import jax.numpy as jnp
from jax.experimental import pallas as pl
from jax.experimental.pallas import tpu as pltpu

_STACK_ROWS = 512
_MOLE_ROWS = 1024


_NSPLIT = 2


def _stack_body(x_ref, Win_ref, bin_ref, decay_ref, Wout_ref, bout_ref,
                gamma_ref, beta_ref, o_ref, Win_s, Wout_s):
    layers = Win_ref.shape[0]

    @pl.when(pl.program_id(0) == 0)
    def _cast_weights():
        Win_s[...] = Win_ref[...].astype(jnp.bfloat16)
        Wout_s[...] = Wout_ref[...].astype(jnp.bfloat16)

    sub = x_ref.shape[0] // _NSPLIT
    hs = [x_ref[pl.ds(j * sub, sub), :] for j in range(_NSPLIT)]
    for i in range(layers):
        gs = [jax.nn.sigmoid(
                  jnp.dot(h.astype(jnp.bfloat16), Win_s[i],
                          preferred_element_type=jnp.float32)
                  + bin_ref[i]) for h in hs]
        nss = [(g * h) * decay_ref[i] for g, h in zip(gs, hs)]
        outs = [jnp.dot(ns.astype(jnp.bfloat16), Wout_s[i],
                        preferred_element_type=jnp.float32)
                + bout_ref[i] for ns in nss]
        ys = [out + h for out, h in zip(outs, hs)]
        new_hs = []
        for y in ys:
            mu = jnp.mean(y, axis=-1, keepdims=True)
            var = jnp.mean((y - mu) ** 2, axis=-1, keepdims=True)
            new_hs.append((y - mu) * jax.lax.rsqrt(var + 1e-5)
                          * gamma_ref[i] + beta_ref[i])
        hs = new_hs
    for j in range(_NSPLIT):
        o_ref[pl.ds(j * sub, sub), :] = hs[j]


def _liquid_stack(x2d, Win, bin_, decay, Wout, bout, gamma, beta):
    rows, d = x2d.shape
    layers = Win.shape[0]
    vec = lambda a: a.reshape(layers, 1, d)
    full = lambda *shape: pl.BlockSpec(shape, lambda r: (0,) * len(shape))
    return pl.pallas_call(
        _stack_body,
        grid=(rows // _STACK_ROWS,),
        in_specs=[
            pl.BlockSpec((_STACK_ROWS, d), lambda r: (r, 0)),
            full(layers, d, d),
            full(layers, 1, d),
            full(layers, 1, d),
            full(layers, d, d),
            full(layers, 1, d),
            full(layers, 1, d),
            full(layers, 1, d),
        ],
        out_specs=pl.BlockSpec((_STACK_ROWS, d), lambda r: (r, 0)),
        out_shape=jax.ShapeDtypeStruct((rows, d), jnp.float32),
        scratch_shapes=[
            pltpu.VMEM((layers, d, d), jnp.bfloat16),
            pltpu.VMEM((layers, d, d), jnp.bfloat16),
        ],
        compiler_params=pltpu.CompilerParams(
            dimension_semantics=("arbitrary",)),
    )(x2d, Win, vec(bin_), vec(decay), Wout, vec(bout), vec(gamma), vec(beta))


def _mole_body(x_ref, Wg_ref, bg_ref, We_ref, be_ref, out_ref, gp_ref, We_s):
    experts = We_ref.shape[0]

    @pl.when(pl.program_id(0) == 0)
    def _cast_weights():
        We_s[...] = We_ref[...].astype(jnp.bfloat16)

    x = x_ref[...]
    xb = x.astype(jnp.bfloat16)
    logits = (jnp.dot(x, Wg_ref[...], preferred_element_type=jnp.float32)
              + bg_ref[...])
    p = jax.nn.softmax(logits, axis=-1)
    acc = jnp.zeros(out_ref.shape, jnp.float32)
    for e in range(experts):
        eo = (jnp.dot(xb, We_s[e], preferred_element_type=jnp.float32)
              + be_ref[e])
        acc = acc + p[:, e][:, None] * eo
    out_ref[...] = acc
    gp_ref[...] = p


def _mole(x2d, Wg, bg, We, be):
    rows, d = x2d.shape
    experts = We.shape[0]
    full = lambda *shape: pl.BlockSpec(shape, lambda r: (0,) * len(shape))
    return pl.pallas_call(
        _mole_body,
        grid=(rows // _MOLE_ROWS,),
        in_specs=[
            pl.BlockSpec((_MOLE_ROWS, d), lambda r: (r, 0)),
            full(d, experts),
            full(1, experts),
            full(experts, d, d),
            full(experts, 1, d),
        ],
        out_specs=[
            pl.BlockSpec((_MOLE_ROWS, d), lambda r: (r, 0)),
            pl.BlockSpec((_MOLE_ROWS, experts), lambda r: (r, 0)),
        ],
        out_shape=[
            jax.ShapeDtypeStruct((rows, d), jnp.float32),
            jax.ShapeDtypeStruct((rows, experts), jnp.float32),
        ],
        scratch_shapes=[pltpu.VMEM((experts, d, d), jnp.bfloat16)],
        compiler_params=pltpu.CompilerParams(
            dimension_semantics=("arbitrary",)),
    )(x2d, Wg, bg.reshape(1, experts), We, be.reshape(experts, 1, d))


def kernel(x_context, x_target,
           enc_Win, enc_bin, enc_decay, enc_Wout, enc_bout, enc_gamma, enc_beta,
           tgt_Win, tgt_bin, tgt_decay, tgt_Wout, tgt_bout, tgt_gamma, tgt_beta,
           Wg, bg, We, be):
    b, s, d = x_context.shape
    xc = x_context.reshape(b * s, d)
    xt = x_target.reshape(b * s, d)
    z_context = _liquid_stack(xc, enc_Win, enc_bin, enc_decay,
                              enc_Wout, enc_bout, enc_gamma, enc_beta)
    z_target = _liquid_stack(xt, tgt_Win, tgt_bin, tgt_decay,
                             tgt_Wout, tgt_bout, tgt_gamma, tgt_beta)
    pred_z, gate_probs = _mole(z_context, Wg, bg, We, be)
    experts = We.shape[0]
    return (pred_z.reshape(b, s, d),
            gate_probs.reshape(b, s, experts),
            z_target.reshape(b, s, d))


# speedup vs baseline: 1.0229x; 1.0229x over previous
import jax
import jax.numpy as jnp
from jax.experimental import pallas as pl
from jax.experimental.pallas import tpu as pltpu

_STACK_ROWS = 1024
_CTX_ROWS = 512
_NSPLIT = 2


def _liquid_layers(hs, Win_ref, bin_ref, decay_ref, Wout_ref, bout_ref,
                   gamma_ref, beta_ref):
    layers = Win_ref.shape[0]
    for i in range(layers):
        gs = [jax.nn.sigmoid(
                  jnp.dot(h, Win_ref[i], preferred_element_type=jnp.float32)
                  + bin_ref[i]) for h in hs]
        nss = [(g * h) * decay_ref[i] for g, h in zip(gs, hs)]
        outs = [jnp.dot(ns, Wout_ref[i], preferred_element_type=jnp.float32)
                + bout_ref[i] for ns in nss]
        ys = [out + h for out, h in zip(outs, hs)]
        new_hs = []
        for y in ys:
            mu = jnp.mean(y, axis=-1, keepdims=True)
            var = jnp.mean((y - mu) ** 2, axis=-1, keepdims=True)
            new_hs.append((y - mu) * jax.lax.rsqrt(var + 1e-5)
                          * gamma_ref[i] + beta_ref[i])
        hs = new_hs
    return hs


def _stack_body(x_ref, Win_ref, bin_ref, decay_ref, Wout_ref, bout_ref,
                gamma_ref, beta_ref, o_ref):
    sub = x_ref.shape[0] // _NSPLIT
    hs = [x_ref[pl.ds(j * sub, sub), :] for j in range(_NSPLIT)]
    hs = _liquid_layers(hs, Win_ref, bin_ref, decay_ref, Wout_ref, bout_ref,
                        gamma_ref, beta_ref)
    for j in range(_NSPLIT):
        o_ref[pl.ds(j * sub, sub), :] = hs[j]


def _liquid_stack(x2d, Win, bin_, decay, Wout, bout, gamma, beta):
    rows, d = x2d.shape
    layers = Win.shape[0]
    vec = lambda a: a.reshape(layers, 1, d)
    full = lambda *shape: pl.BlockSpec(shape, lambda r: (0,) * len(shape))
    return pl.pallas_call(
        _stack_body,
        grid=(rows // _STACK_ROWS,),
        in_specs=[
            pl.BlockSpec((_STACK_ROWS, d), lambda r: (r, 0)),
            full(layers, d, d),
            full(layers, 1, d),
            full(layers, 1, d),
            full(layers, d, d),
            full(layers, 1, d),
            full(layers, 1, d),
            full(layers, 1, d),
        ],
        out_specs=pl.BlockSpec((_STACK_ROWS, d), lambda r: (r, 0)),
        out_shape=jax.ShapeDtypeStruct((rows, d), jnp.float32),
        compiler_params=pltpu.CompilerParams(
            dimension_semantics=("arbitrary",)),
    )(x2d, Win, vec(bin_), vec(decay), Wout, vec(bout), vec(gamma), vec(beta))


def _ctx_body(x_ref, Win_ref, bin_ref, decay_ref, Wout_ref, bout_ref,
              gamma_ref, beta_ref, Wg_ref, bg_ref, We_ref, be_ref,
              out_ref, gp_ref):
    experts = We_ref.shape[0]
    sub = x_ref.shape[0] // _NSPLIT
    hs = [x_ref[pl.ds(j * sub, sub), :] for j in range(_NSPLIT)]
    hs = _liquid_layers(hs, Win_ref, bin_ref, decay_ref, Wout_ref, bout_ref,
                        gamma_ref, beta_ref)
    for j, z in enumerate(hs):
        logits = (jnp.dot(z, Wg_ref[...], preferred_element_type=jnp.float32)
                  + bg_ref[...])
        p = jax.nn.softmax(logits, axis=-1)
        acc = jnp.zeros((sub, out_ref.shape[1]), jnp.float32)
        for e in range(experts):
            eo = (jnp.dot(z, We_ref[e], preferred_element_type=jnp.float32)
                  + be_ref[e])
            acc = acc + p[:, e][:, None] * eo
        out_ref[pl.ds(j * sub, sub), :] = acc
        gp_ref[pl.ds(j * sub, sub), :] = p


def _ctx_stack_mole(x2d, Win, bin_, decay, Wout, bout, gamma, beta,
                    Wg, bg, We, be):
    rows, d = x2d.shape
    layers = Win.shape[0]
    experts = We.shape[0]
    vec = lambda a: a.reshape(layers, 1, d)
    full = lambda *shape: pl.BlockSpec(shape, lambda r: (0,) * len(shape))
    return pl.pallas_call(
        _ctx_body,
        grid=(rows // _CTX_ROWS,),
        in_specs=[
            pl.BlockSpec((_CTX_ROWS, d), lambda r: (r, 0)),
            full(layers, d, d),
            full(layers, 1, d),
            full(layers, 1, d),
            full(layers, d, d),
            full(layers, 1, d),
            full(layers, 1, d),
            full(layers, 1, d),
            full(d, experts),
            full(1, experts),
            full(experts, d, d),
            full(experts, 1, d),
        ],
        out_specs=[
            pl.BlockSpec((_CTX_ROWS, d), lambda r: (r, 0)),
            pl.BlockSpec((_CTX_ROWS, experts), lambda r: (r, 0)),
        ],
        out_shape=[
            jax.ShapeDtypeStruct((rows, d), jnp.float32),
            jax.ShapeDtypeStruct((rows, experts), jnp.float32),
        ],
        compiler_params=pltpu.CompilerParams(
            dimension_semantics=("arbitrary",)),
    )(x2d, Win, vec(bin_), vec(decay), Wout, vec(bout), vec(gamma),
      vec(beta), Wg, bg.reshape(1, experts), We, be.reshape(experts, 1, d))


def kernel(x_context, x_target,
           enc_Win, enc_bin, enc_decay, enc_Wout, enc_bout, enc_gamma, enc_beta,
           tgt_Win, tgt_bin, tgt_decay, tgt_Wout, tgt_bout, tgt_gamma, tgt_beta,
           Wg, bg, We, be):
    b, s, d = x_context.shape
    experts = We.shape[0]
    xc = x_context.reshape(b * s, d)
    xt = x_target.reshape(b * s, d)
    pred_z, gate_probs = _ctx_stack_mole(
        xc, enc_Win, enc_bin, enc_decay, enc_Wout, enc_bout, enc_gamma,
        enc_beta, Wg, bg, We, be)
    z_target = _liquid_stack(xt, tgt_Win, tgt_bin, tgt_decay,
                             tgt_Wout, tgt_bout, tgt_gamma, tgt_beta)
    return (pred_z.reshape(b, s, d),
            gate_probs.reshape(b, s, experts),
            z_target.reshape(b, s, d))


# speedup vs baseline: 1.0237x; 1.0008x over previous
import jax
import jax.numpy as jnp
from jax.experimental import pallas as pl
from jax.experimental.pallas import tpu as pltpu

_STACK_ROWS = 1024
_CTX_ROWS = 512
_NSPLIT = 2


def _liquid_layers(hs, Win_ref, bin_ref, decay_ref, Wout_ref, bout_ref,
                   gamma_ref, beta_ref):
    layers = Win_ref.shape[0]
    for i in range(layers):
        gs = [0.5 * jnp.tanh(
                  0.5 * (jnp.dot(h, Win_ref[i],
                                 preferred_element_type=jnp.float32)
                         + bin_ref[i])) + 0.5 for h in hs]
        nss = [(g * h) * decay_ref[i] for g, h in zip(gs, hs)]
        outs = [jnp.dot(ns, Wout_ref[i], preferred_element_type=jnp.float32)
                + bout_ref[i] for ns in nss]
        ys = [out + h for out, h in zip(outs, hs)]
        new_hs = []
        for y in ys:
            mu = jnp.mean(y, axis=-1, keepdims=True)
            var = jnp.mean(y * y, axis=-1, keepdims=True) - mu * mu
            a = jax.lax.rsqrt(var + 1e-5) * gamma_ref[i]
            new_hs.append(y * a + (beta_ref[i] - mu * a))
        hs = new_hs
    return hs


def _stack_body(x_ref, Win_ref, bin_ref, decay_ref, Wout_ref, bout_ref,
                gamma_ref, beta_ref, o_ref):
    sub = x_ref.shape[0] // _NSPLIT
    hs = [x_ref[pl.ds(j * sub, sub), :] for j in range(_NSPLIT)]
    hs = _liquid_layers(hs, Win_ref, bin_ref, decay_ref, Wout_ref, bout_ref,
                        gamma_ref, beta_ref)
    for j in range(_NSPLIT):
        o_ref[pl.ds(j * sub, sub), :] = hs[j]


def _liquid_stack(x2d, Win, bin_, decay, Wout, bout, gamma, beta):
    rows, d = x2d.shape
    layers = Win.shape[0]
    vec = lambda a: a.reshape(layers, 1, d)
    full = lambda *shape: pl.BlockSpec(shape, lambda r: (0,) * len(shape))
    return pl.pallas_call(
        _stack_body,
        grid=(rows // _STACK_ROWS,),
        in_specs=[
            pl.BlockSpec((_STACK_ROWS, d), lambda r: (r, 0)),
            full(layers, d, d),
            full(layers, 1, d),
            full(layers, 1, d),
            full(layers, d, d),
            full(layers, 1, d),
            full(layers, 1, d),
            full(layers, 1, d),
        ],
        out_specs=pl.BlockSpec((_STACK_ROWS, d), lambda r: (r, 0)),
        out_shape=jax.ShapeDtypeStruct((rows, d), jnp.float32),
        compiler_params=pltpu.CompilerParams(
            dimension_semantics=("arbitrary",)),
    )(x2d, Win, vec(bin_), vec(decay), Wout, vec(bout), vec(gamma), vec(beta))


def _ctx_body(x_ref, Win_ref, bin_ref, decay_ref, Wout_ref, bout_ref,
              gamma_ref, beta_ref, Wg_ref, bg_ref, We_ref, be_ref,
              out_ref, gp_ref):
    experts = We_ref.shape[0]
    sub = x_ref.shape[0] // _NSPLIT
    hs = [x_ref[pl.ds(j * sub, sub), :] for j in range(_NSPLIT)]
    hs = _liquid_layers(hs, Win_ref, bin_ref, decay_ref, Wout_ref, bout_ref,
                        gamma_ref, beta_ref)
    for j, z in enumerate(hs):
        logits = (jnp.dot(z, Wg_ref[...], preferred_element_type=jnp.float32)
                  + bg_ref[...])
        p = jax.nn.softmax(logits, axis=-1)
        acc = jnp.zeros((sub, out_ref.shape[1]), jnp.float32)
        for e in range(experts):
            eo = (jnp.dot(z, We_ref[e], preferred_element_type=jnp.float32)
                  + be_ref[e])
            acc = acc + p[:, e][:, None] * eo
        out_ref[pl.ds(j * sub, sub), :] = acc
        gp_ref[pl.ds(j * sub, sub), :] = p


def _ctx_stack_mole(x2d, Win, bin_, decay, Wout, bout, gamma, beta,
                    Wg, bg, We, be):
    rows, d = x2d.shape
    layers = Win.shape[0]
    experts = We.shape[0]
    vec = lambda a: a.reshape(layers, 1, d)
    full = lambda *shape: pl.BlockSpec(shape, lambda r: (0,) * len(shape))
    return pl.pallas_call(
        _ctx_body,
        grid=(rows // _CTX_ROWS,),
        in_specs=[
            pl.BlockSpec((_CTX_ROWS, d), lambda r: (r, 0)),
            full(layers, d, d),
            full(layers, 1, d),
            full(layers, 1, d),
            full(layers, d, d),
            full(layers, 1, d),
            full(layers, 1, d),
            full(layers, 1, d),
            full(d, experts),
            full(1, experts),
            full(experts, d, d),
            full(experts, 1, d),
        ],
        out_specs=[
            pl.BlockSpec((_CTX_ROWS, d), lambda r: (r, 0)),
            pl.BlockSpec((_CTX_ROWS, experts), lambda r: (r, 0)),
        ],
        out_shape=[
            jax.ShapeDtypeStruct((rows, d), jnp.float32),
            jax.ShapeDtypeStruct((rows, experts), jnp.float32),
        ],
        compiler_params=pltpu.CompilerParams(
            dimension_semantics=("arbitrary",)),
    )(x2d, Win, vec(bin_), vec(decay), Wout, vec(bout), vec(gamma),
      vec(beta), Wg, bg.reshape(1, experts), We, be.reshape(experts, 1, d))


def kernel(x_context, x_target,
           enc_Win, enc_bin, enc_decay, enc_Wout, enc_bout, enc_gamma, enc_beta,
           tgt_Win, tgt_bin, tgt_decay, tgt_Wout, tgt_bout, tgt_gamma, tgt_beta,
           Wg, bg, We, be):
    b, s, d = x_context.shape
    experts = We.shape[0]
    xc = x_context.reshape(b * s, d)
    xt = x_target.reshape(b * s, d)
    pred_z, gate_probs = _ctx_stack_mole(
        xc, enc_Win, enc_bin, enc_decay, enc_Wout, enc_bout, enc_gamma,
        enc_beta, Wg, bg, We, be)
    z_target = _liquid_stack(xt, tgt_Win, tgt_bin, tgt_decay,
                             tgt_Wout, tgt_bout, tgt_gamma, tgt_beta)
    return (pred_z.reshape(b, s, d),
            gate_probs.reshape(b, s, experts),
            z_target.reshape(b, s, d))
